# baseline (speedup 1.0000x reference)
import jax
import jax.numpy as jnp
from jax.experimental import pallas as pl

_MASK_PCT = 0.8


def _make_field_idx(bs: int):
    n_masked = int(_MASK_PCT * bs)
    jkey = jax.random.key(42)
    j = jax.random.randint(jkey, (n_masked,), 0, 5, dtype=jnp.int32)
    return jnp.concatenate([j, -jnp.ones((bs - n_masked,), dtype=jnp.int32)])


def _mask_kernel(j_ref, dgb_ref, prb_ref, odb_ref, dgp_ref, prp_ref, odp_ref,
                 age_ref, gen_ref,
                 o_dgb, o_prb, o_odb, o_dgp, o_prp, o_odp, o_age, o_gen):
    j = j_ref[...]
    keep0 = (j != 0)
    keep1 = (j != 1)
    keep2 = (j != 2)
    z = jnp.float32(0.0)
    o_dgb[...] = jnp.where(keep0, dgb_ref[...], z)
    o_dgp[...] = jnp.where(keep0, dgp_ref[...], z)
    o_prb[...] = jnp.where(keep1, prb_ref[...], z)
    o_prp[...] = jnp.where(keep1, prp_ref[...], z)
    o_odb[...] = jnp.where(keep2, odb_ref[...], z)
    o_odp[...] = jnp.where(keep2, odp_ref[...], z)
    o_age[...] = jnp.where(j != 3, age_ref[...], z)
    o_gen[...] = jnp.where(j != 4, gen_ref[...], z)


def kernel(x_dg_bin, x_prod_bin, x_odb_bin, x_dg_pe, x_prod_pe, x_odb_pe,
           x_age, x_gender):
    B, D = x_dg_bin.shape
    j = _make_field_idx(B).reshape(B, 1)
    age2 = x_age.reshape(B, 1)
    gen2 = x_gender.reshape(B, 1)

    bm = 2048
    grid = (B // bm,)
    big = pl.BlockSpec((bm, D), lambda i: (i, 0))
    vec = pl.BlockSpec((bm, 1), lambda i: (i, 0))

    big_t = jax.ShapeDtypeStruct((B, D), jnp.float32)
    vec_t = jax.ShapeDtypeStruct((B, 1), jnp.float32)

    outs = pl.pallas_call(
        _mask_kernel,
        grid=grid,
        in_specs=[vec, big, big, big, big, big, big, vec, vec],
        out_specs=[big, big, big, big, big, big, vec, vec],
        out_shape=[big_t, big_t, big_t, big_t, big_t, big_t, vec_t, vec_t],
    )(j, x_dg_bin, x_prod_bin, x_odb_bin, x_dg_pe, x_prod_pe, x_odb_pe,
      age2, gen2)

    (o_dgb, o_prb, o_odb, o_dgp, o_prp, o_odp, o_age, o_gen) = outs
    return (o_dgb, o_prb, o_odb, o_dgp, o_prp, o_odp,
            o_age.reshape(B), o_gen.reshape(B))


# speedup vs baseline: 1.8779x; 1.8779x over previous
import jax
import jax.numpy as jnp
from jax.experimental import pallas as pl

_MASK_PCT = 0.8


def _make_field_idx(bs: int):
    n_masked = int(_MASK_PCT * bs)
    jkey = jax.random.key(42)
    j = jax.random.randint(jkey, (n_masked,), 0, 5, dtype=jnp.int32)
    return jnp.concatenate([j, -jnp.ones((bs - n_masked,), dtype=jnp.int32)])


def _mask_kernel(j_ref, dgb_ref, prb_ref, odb_ref, dgp_ref, prp_ref, odp_ref,
                 age_ref, gen_ref,
                 o_dgb, o_prb, o_odb, o_dgp, o_prp, o_odp, o_age, o_gen):
    j = j_ref[...]
    z = jnp.float32(0.0)
    keep0 = (j != 0).astype(jnp.float32)[:, :, None]
    keep1 = (j != 1).astype(jnp.float32)[:, :, None]
    keep2 = (j != 2).astype(jnp.float32)[:, :, None]
    o_dgb[...] = dgb_ref[...] * keep0
    o_dgp[...] = dgp_ref[...] * keep0
    o_prb[...] = prb_ref[...] * keep1
    o_prp[...] = prp_ref[...] * keep1
    o_odb[...] = odb_ref[...] * keep2
    o_odp[...] = odp_ref[...] * keep2
    o_age[...] = jnp.where(j != 3, age_ref[...], z)
    o_gen[...] = jnp.where(j != 4, gen_ref[...], z)


def kernel(x_dg_bin, x_prod_bin, x_odb_bin, x_dg_pe, x_prod_pe, x_odb_pe,
           x_age, x_gender):
    B, D = x_dg_bin.shape
    R = B // 128
    j = _make_field_idx(B).reshape(R, 128)
    age2 = x_age.reshape(R, 128)
    gen2 = x_gender.reshape(R, 128)
    bigs = [x.reshape(R, 128, D) for x in
            (x_dg_bin, x_prod_bin, x_odb_bin, x_dg_pe, x_prod_pe, x_odb_pe)]

    bm = 16
    grid = (R // bm,)
    big = pl.BlockSpec((bm, 128, D), lambda i: (i, 0, 0))
    vec = pl.BlockSpec((bm, 128), lambda i: (i, 0))

    big_t = jax.ShapeDtypeStruct((R, 128, D), jnp.float32)
    vec_t = jax.ShapeDtypeStruct((R, 128), jnp.float32)

    outs = pl.pallas_call(
        _mask_kernel,
        grid=grid,
        in_specs=[vec, big, big, big, big, big, big, vec, vec],
        out_specs=[big, big, big, big, big, big, vec, vec],
        out_shape=[big_t, big_t, big_t, big_t, big_t, big_t, vec_t, vec_t],
    )(j, *bigs, age2, gen2)

    (o_dgb, o_prb, o_odb, o_dgp, o_prp, o_odp, o_age, o_gen) = outs
    return (o_dgb.reshape(B, D), o_prb.reshape(B, D), o_odb.reshape(B, D),
            o_dgp.reshape(B, D), o_prp.reshape(B, D), o_odp.reshape(B, D),
            o_age.reshape(B), o_gen.reshape(B))


# speedup vs baseline: 1.9554x; 1.0413x over previous
import jax
import jax.numpy as jnp
from jax.experimental import pallas as pl

_MASK_PCT = 0.8


def _make_field_idx(bs: int):
    n_masked = int(_MASK_PCT * bs)
    jkey = jax.random.key(42)
    j = jax.random.randint(jkey, (n_masked,), 0, 5, dtype=jnp.int32)
    return jnp.concatenate([j, -jnp.ones((bs - n_masked,), dtype=jnp.int32)])


def _mask_kernel(j_ref, dgb_ref, prb_ref, odb_ref, dgp_ref, prp_ref, odp_ref,
                 age_ref, gen_ref,
                 o_dgb, o_prb, o_odb, o_dgp, o_prp, o_odp, o_age, o_gen):
    j = j_ref[...]
    z = jnp.float32(0.0)
    keep0 = (j != 0).astype(jnp.float32)[:, :, None]
    keep1 = (j != 1).astype(jnp.float32)[:, :, None]
    keep2 = (j != 2).astype(jnp.float32)[:, :, None]
    o_dgb[...] = dgb_ref[...] * keep0
    o_dgp[...] = dgp_ref[...] * keep0
    o_prb[...] = prb_ref[...] * keep1
    o_prp[...] = prp_ref[...] * keep1
    o_odb[...] = odb_ref[...] * keep2
    o_odp[...] = odp_ref[...] * keep2
    o_age[...] = jnp.where(j != 3, age_ref[...], z)
    o_gen[...] = jnp.where(j != 4, gen_ref[...], z)


def kernel(x_dg_bin, x_prod_bin, x_odb_bin, x_dg_pe, x_prod_pe, x_odb_pe,
           x_age, x_gender):
    B, D = x_dg_bin.shape
    R = B // 128
    j = _make_field_idx(B).reshape(R, 128)
    age2 = x_age.reshape(R, 128)
    gen2 = x_gender.reshape(R, 128)
    bigs = [x.reshape(R, 128, D) for x in
            (x_dg_bin, x_prod_bin, x_odb_bin, x_dg_pe, x_prod_pe, x_odb_pe)]

    bm = 32
    grid = (R // bm,)
    big = pl.BlockSpec((bm, 128, D), lambda i: (i, 0, 0))
    vec = pl.BlockSpec((bm, 128), lambda i: (i, 0))

    big_t = jax.ShapeDtypeStruct((R, 128, D), jnp.float32)
    vec_t = jax.ShapeDtypeStruct((R, 128), jnp.float32)

    outs = pl.pallas_call(
        _mask_kernel,
        grid=grid,
        in_specs=[vec, big, big, big, big, big, big, vec, vec],
        out_specs=[big, big, big, big, big, big, vec, vec],
        out_shape=[big_t, big_t, big_t, big_t, big_t, big_t, vec_t, vec_t],
    )(j, *bigs, age2, gen2)

    (o_dgb, o_prb, o_odb, o_dgp, o_prp, o_odp, o_age, o_gen) = outs
    return (o_dgb.reshape(B, D), o_prb.reshape(B, D), o_odb.reshape(B, D),
            o_dgp.reshape(B, D), o_prp.reshape(B, D), o_odp.reshape(B, D),
            o_age.reshape(B), o_gen.reshape(B))
